# Initial kernel scaffold; baseline (speedup 1.0000x reference)
import jax
import jax.numpy as jnp
from jax.experimental import pallas as pl


def kernel(in_feat, edge_index, W1, b1, W2, b2, W3, b3, W4, b4, W5, b5, Wl, bl):
    raise NotImplementedError("write your pallas kernel here")


# speedup vs baseline: 3.1344x; 3.1344x over previous
import functools

import jax
import jax.numpy as jnp
from jax import lax
from jax.experimental import pallas as pl
from jax.experimental.pallas import tpu as pltpu
from jax.experimental.pallas import tpu_sc as plsc

_N = 10000
_E = 320000
_D = 128
_C = 16

_NC = 2
_NS = 16
_NW = _NC * _NS
_K = 128
_CPT = 80
_EPAD = _NW * _CPT * _K
_NP = 10240
_TRASH = 10200
_RPT = _NP // _NS
_NBUF = 2
_XR = 4

_mesh = plsc.VectorSubcoreMesh(
    core_axis_name="c", subcore_axis_name="s", num_cores=_NC, num_subcores=_NS
)


def _idx_pair_start(src_hbm, dst_hbm, si_ring, di_ring, sem, base, i, q):
    e0 = pl.multiple_of((base + i) * _K, 8)
    pltpu.async_copy(src_hbm.at[pl.ds(e0, _K)], si_ring.at[q], sem)
    pltpu.async_copy(dst_hbm.at[pl.ds(e0, _K)], di_ring.at[q], sem)


def _idx_pair_wait(src_hbm, dst_hbm, si_ring, di_ring, sem, base, i, q):
    e0 = pl.multiple_of((base + i) * _K, 8)
    pltpu.make_async_copy(src_hbm.at[pl.ds(e0, _K)], si_ring.at[q], sem).wait()
    pltpu.make_async_copy(dst_hbm.at[pl.ds(e0, _K)], di_ring.at[q], sem).wait()


@functools.partial(
    pl.kernel,
    out_type=jax.ShapeDtypeStruct((_NC, _NP, _D), jnp.float32),
    mesh=_mesh,
    scratch_types=[
        pltpu.VMEM((_XR, _K), jnp.int32),
        pltpu.VMEM((_XR, _K), jnp.int32),
        pltpu.VMEM((_K, _D), jnp.float32),
        pltpu.VMEM((_K, _D), jnp.float32),
        pltpu.VMEM_SHARED((_NP, _D), jnp.float32),
        [pltpu.SemaphoreType.DMA] * _XR,
        [pltpu.SemaphoreType.DMA] * _XR,
    ],
)
def _degree_kernel(src_hbm, dst_hbm, zeros_hbm, ones_s_hbm, ones_d_hbm,
                   out_hbm, si_ring, di_ring, ones_s, ones_d, acc, xsems,
                   ssems):
    c = lax.axis_index("c")
    s = lax.axis_index("s")
    wid = c * _NS + s
    base = wid * _CPT
    r0 = s * _RPT
    pltpu.sync_copy(ones_s_hbm, ones_s)
    pltpu.sync_copy(ones_d_hbm, ones_d)
    pltpu.sync_copy(zeros_hbm.at[pl.ds(r0, _RPT)], acc.at[pl.ds(r0, _RPT)])
    plsc.subcore_barrier()

    for q in range(_XR):
        _idx_pair_start(src_hbm, dst_hbm, si_ring, di_ring, xsems[q],
                        base, q, q)

    def slot(i, q):
        _idx_pair_wait(src_hbm, dst_hbm, si_ring, di_ring, xsems[q],
                       base, i, q)
        pltpu.async_copy(ones_s, acc.at[si_ring.at[q]], ssems[q], add=True)
        sd = pltpu.async_copy(ones_d, acc.at[di_ring.at[q]], ssems[q],
                              add=True)
        pltpu.make_async_copy(ones_s, acc.at[si_ring.at[q]], ssems[q]).wait()
        sd.wait()

    def body(i0, carry):
        for b in range(_XR):
            i = i0 * _XR + b
            slot(i, b)
            _idx_pair_start(src_hbm, dst_hbm, si_ring, di_ring, xsems[b],
                            base, i + _XR, b)
        return carry

    lax.fori_loop(0, _CPT // _XR - 1, body, 0)
    for b in range(_XR):
        slot(_CPT - _XR + b, b)

    plsc.subcore_barrier()
    pltpu.sync_copy(acc.at[pl.ds(r0, _RPT)],
                    out_hbm.at[c].at[pl.ds(r0, _RPT)])


@functools.partial(
    pl.kernel,
    out_type=jax.ShapeDtypeStruct((_NC, _NP, _D), jnp.float32),
    mesh=_mesh,
    scratch_types=[
        pltpu.VMEM((_XR, _K), jnp.int32),
        pltpu.VMEM((_XR, _K), jnp.int32),
        pltpu.VMEM((_NBUF, _K, _D), jnp.float32),
        pltpu.VMEM_SHARED((_NP, _D), jnp.float32),
        [pltpu.SemaphoreType.DMA] * _XR,
        [pltpu.SemaphoreType.DMA] * _NBUF,
    ],
)
def _agg_kernel(h_hbm, src_hbm, dst_hbm, zeros_hbm, out_hbm, si_ring, di_ring,
                rows_v, acc_sh, xsems, gsems):
    c = lax.axis_index("c")
    s = lax.axis_index("s")
    wid = c * _NS + s
    base = wid * _CPT
    r0 = s * _RPT
    pltpu.sync_copy(zeros_hbm.at[pl.ds(r0, _RPT)], acc_sh.at[pl.ds(r0, _RPT)])
    plsc.subcore_barrier()

    for q in range(_XR):
        _idx_pair_start(src_hbm, dst_hbm, si_ring, di_ring, xsems[q],
                        base, q, q)
    for b in range(_NBUF):
        _idx_pair_wait(src_hbm, dst_hbm, si_ring, di_ring, xsems[b],
                       base, b, b)
        pltpu.async_copy(h_hbm.at[si_ring.at[b]], rows_v.at[b], gsems[b])

    def slot(i, b, q, q2):
        pltpu.make_async_copy(
            h_hbm.at[si_ring.at[q]], rows_v.at[b], gsems[b]).wait()
        pltpu.sync_copy(rows_v.at[b], acc_sh.at[di_ring.at[q]], add=True)

    def body(i0, carry):
        for b in range(_XR):
            i = i0 * _XR + b
            rb = b % _NBUF
            q2 = (b + _NBUF) % _XR
            slot(i, rb, b, q2)
            _idx_pair_start(src_hbm, dst_hbm, si_ring, di_ring, xsems[b],
                            base, i + _XR, b)
            _idx_pair_wait(src_hbm, dst_hbm, si_ring, di_ring, xsems[q2],
                           base, i + _NBUF, q2)
            pltpu.async_copy(
                h_hbm.at[si_ring.at[q2]], rows_v.at[rb], gsems[rb])
        return carry

    lax.fori_loop(0, _CPT // _XR - 1, body, 0)
    for b in range(_XR):
        i = _CPT - _XR + b
        rb = b % _NBUF
        q2 = (b + _NBUF) % _XR
        slot(i, rb, b, q2)
        if b < _XR - _NBUF:
            _idx_pair_wait(src_hbm, dst_hbm, si_ring, di_ring, xsems[q2],
                           base, i + _NBUF, q2)
            pltpu.async_copy(
                h_hbm.at[si_ring.at[q2]], rows_v.at[rb], gsems[rb])

    plsc.subcore_barrier()
    pltpu.sync_copy(acc_sh.at[pl.ds(r0, _RPT)],
                    out_hbm.at[c].at[pl.ds(r0, _RPT)])


_NB = 2000
_GRID = _N // _NB


def _norm_body(deg_ref, x_ref, ns_ref, nd_ref, h0_ref):
    p = deg_ref[...]
    ds = p[0, :, 0] + p[1, :, 0]
    di = p[0, :, 64] + p[1, :, 64]
    ns = jnp.where(ds > 0, lax.rsqrt(ds), 0.0)[:, None]
    nd = jnp.where(di > 0, lax.rsqrt(di), 0.0)[:, None]
    ns_ref[...] = ns
    nd_ref[...] = nd
    h0_ref[...] = x_ref[...] * ns


def _norm_stage(deg_parts, x):
    return pl.pallas_call(
        _norm_body,
        grid=(_GRID,),
        in_specs=[
            pl.BlockSpec((_NC, _NB, _D), lambda i: (0, i, 0)),
            pl.BlockSpec((_NB, _D), lambda i: (i, 0)),
        ],
        out_specs=[
            pl.BlockSpec((_NB, 1), lambda i: (i, 0)),
            pl.BlockSpec((_NB, 1), lambda i: (i, 0)),
            pl.BlockSpec((_NB, _D), lambda i: (i, 0)),
        ],
        out_shape=[
            jax.ShapeDtypeStruct((_N, 1), jnp.float32),
            jax.ShapeDtypeStruct((_N, 1), jnp.float32),
            jax.ShapeDtypeStruct((_N, _D), jnp.float32),
        ],
    )(deg_parts, x)


def _layer_body(p_ref, nd_ref, w_ref, b_ref, ns_ref, o_ref):
    a = (p_ref[0] + p_ref[1]) * nd_ref[...]
    y = jnp.dot(a, w_ref[...], preferred_element_type=jnp.float32) + b_ref[...]
    y = jnp.where(y > 0, y, 0.01 * y)
    o_ref[...] = y * ns_ref[...]


def _layer_stage(parts, nd, w, b, ns):
    return pl.pallas_call(
        _layer_body,
        grid=(_GRID,),
        in_specs=[
            pl.BlockSpec((_NC, _NB, _D), lambda i: (0, i, 0)),
            pl.BlockSpec((_NB, 1), lambda i: (i, 0)),
            pl.BlockSpec((_D, _D), lambda i: (0, 0)),
            pl.BlockSpec((1, _D), lambda i: (0, 0)),
            pl.BlockSpec((_NB, 1), lambda i: (i, 0)),
        ],
        out_specs=pl.BlockSpec((_NB, _D), lambda i: (i, 0)),
        out_shape=jax.ShapeDtypeStruct((_N, _D), jnp.float32),
    )(parts, nd, w, b, ns)


def _final_body(p_ref, nd_ref, w_ref, b_ref, wl_ref, bl_ref, o_ref):
    a = (p_ref[0] + p_ref[1]) * nd_ref[...]
    y = jnp.dot(a, w_ref[...], preferred_element_type=jnp.float32) + b_ref[...]
    y = jnp.where(y > 0, y, 0.01 * y)
    o_ref[...] = (
        jnp.dot(y, wl_ref[...], preferred_element_type=jnp.float32) + bl_ref[...]
    )


def _final_stage(parts, nd, w, b, wl, bl):
    return pl.pallas_call(
        _final_body,
        grid=(_GRID,),
        in_specs=[
            pl.BlockSpec((_NC, _NB, _D), lambda i: (0, i, 0)),
            pl.BlockSpec((_NB, 1), lambda i: (i, 0)),
            pl.BlockSpec((_D, _D), lambda i: (0, 0)),
            pl.BlockSpec((1, _D), lambda i: (0, 0)),
            pl.BlockSpec((_D, _C), lambda i: (0, 0)),
            pl.BlockSpec((1, _C), lambda i: (0, 0)),
        ],
        out_specs=pl.BlockSpec((_NB, _C), lambda i: (i, 0)),
        out_shape=jax.ShapeDtypeStruct((_N, _C), jnp.float32),
    )(parts, nd, w, b, wl, bl)


def kernel(in_feat, edge_index, W1, b1, W2, b2, W3, b3, W4, b4, W5, b5, Wl, bl):
    src = edge_index[0]
    dst = edge_index[1]
    npad = _EPAD - _E
    pad0 = jnp.zeros((npad,), jnp.int32)
    padt = jnp.full((npad,), _TRASH, jnp.int32)
    src_agg = jnp.concatenate([src, pad0])
    src_deg = jnp.concatenate([src, padt])
    dst_pad = jnp.concatenate([dst, padt])
    zeros_big = jnp.zeros((_NP, _D), jnp.float32)
    lanes = jnp.arange(_D) < 64
    ones_s = jnp.broadcast_to(lanes.astype(jnp.float32), (_K, _D))
    ones_d = jnp.broadcast_to((~lanes).astype(jnp.float32), (_K, _D))

    deg_parts = _degree_kernel(src_deg, dst_pad, zeros_big, ones_s, ones_d)
    ns, nd, h = _norm_stage(deg_parts, in_feat)
    for w, b in ((W1, b1), (W2, b2), (W3, b3), (W4, b4)):
        parts = _agg_kernel(h, src_agg, dst_pad, zeros_big)
        h = _layer_stage(parts, nd, w, b.reshape(1, _D), ns)
    parts = _agg_kernel(h, src_agg, dst_pad, zeros_big)
    return _final_stage(parts, nd, W5, b5.reshape(1, _D), Wl, bl.reshape(1, _C))
